# baseline (speedup 1.0000x reference)
import functools

import jax
import jax.numpy as jnp
from jax import lax
from jax.experimental import pallas as pl
from jax.experimental.pallas import tpu as pltpu
from jax.experimental.pallas import tpu_sc as plsc

N = 1600000
V = 50000
D_OUT = 64
VX = 0.2
VY = 0.2
X_OFFSET = VX / 2 + 0.0
Y_OFFSET = VY / 2 + (-40.0)

NC = 2
NS = 16
NW = NC * NS
C = N // NW
P = 5000
M = N // 16
BS_ITERS = 17

_mesh = plsc.VectorSubcoreMesh(core_axis_name="c", subcore_axis_name="s")


@functools.partial(
    pl.kernel,
    out_type=jax.ShapeDtypeStruct((V * D_OUT,), jnp.float32),
    mesh=_mesh,
    scratch_types=[
        pltpu.VMEM((P, 4), jnp.float32),
        pltpu.VMEM((P + 32,), jnp.int32),
        pltpu.VMEM((P, 4), jnp.int32),
        pltpu.VMEM((9, 64), jnp.float32),
        pltpu.VMEM((16,), jnp.int32),
        pltpu.VMEM((16,), jnp.int32),
        pltpu.VMEM((16,), jnp.int32),
        pltpu.VMEM((16 * 64,), jnp.float32),
        pltpu.VMEM((16 * 64,), jnp.float32),
        pltpu.SemaphoreType.DMA,
        pltpu.SemaphoreType.DMA,
    ],
    compiler_params=pltpu.CompilerParams(use_tc_tiling_on_sc=False,
                                         needs_layout_passes=False),
)
def _pfnet_kernel(points_hbm, ids_hbm, grid_hbm, wt_hbm, out_hbm,
                  pts_b, ids_b, grd_b, wt_b, lo_b, hi_b, bs_b, stage_b,
                  zero_b, zsem, sem):
    w = lax.axis_index("c") * NS + lax.axis_index("s")

    pltpu.sync_copy(wt_hbm, wt_b)

    @pl.when(w > 0)
    def _():
        pltpu.sync_copy(ids_hbm.at[pl.ds(pl.multiple_of(w * C - 16, 16), 16)],
                        lo_b)

    @pl.when(w == 0)
    def _():
        lo_b[...] = jnp.full((16,), -1, jnp.int32)

    pltpu.sync_copy(
        ids_hbm.at[pl.ds(pl.multiple_of((w + 1) * C - 16, 16), 16)], hi_b)

    own_lo = lo_b[...][15] + 1
    own_hi = jnp.where(w == NW - 1, V, hi_b[...][15] + 1)

    for t in range(64):
        zero_b[pl.ds(16 * t, 16)] = jnp.zeros((16,), jnp.float32)

    nz = own_hi - own_lo
    ntile = lax.div(nz, 16)
    tail = nz - ntile * 16

    def ztile(t, _):
        dst = pl.multiple_of((own_lo + 16 * t) * 64, 64)
        pltpu.async_copy(zero_b, out_hbm.at[pl.ds(dst, 16 * 64)], zsem)
        return 0

    lax.fori_loop(0, ntile, ztile, 0)

    def zrow(j, _):
        dst = pl.multiple_of((own_lo + ntile * 16 + j) * 64, 64)
        pltpu.async_copy(zero_b.at[pl.ds(0, 64)],
                         out_hbm.at[pl.ds(dst, 64)], zsem)
        return 0

    lax.fori_loop(0, tail, zrow, 0)

    def zdrain_t(t, _):
        pltpu.make_async_copy(
            zero_b, out_hbm.at[pl.ds(pl.multiple_of(own_lo * 64, 64),
                                     16 * 64)], zsem).wait()
        return 0

    lax.fori_loop(0, ntile, zdrain_t, 0)

    def zdrain_r(j, _):
        pltpu.make_async_copy(
            zero_b.at[pl.ds(0, 64)],
            out_hbm.at[pl.ds(pl.multiple_of(own_lo * 64, 64), 64)],
            zsem).wait()
        return 0

    lax.fori_loop(0, tail, zdrain_r, 0)

    def bpos(x):
        def bs_body(_, s):
            lo, hi = s
            mid = lax.div(lo + hi, 2)
            pltpu.sync_copy(
                ids_hbm.at[pl.ds(pl.multiple_of(16 * mid, 16), 16)], bs_b)
            ge = bs_b[...][0] >= x
            lo2 = jnp.where(jnp.logical_and(lo < hi, jnp.logical_not(ge)),
                            mid + 1, lo)
            hi2 = jnp.where(jnp.logical_and(lo < hi, ge), mid, hi)
            return lo2, hi2

        jstar, _ = lax.fori_loop(0, BS_ITERS, bs_body,
                                 (jnp.int32(0), jnp.int32(M)))
        jm1 = jnp.maximum(jstar - 1, 0)
        pltpu.sync_copy(
            ids_hbm.at[pl.ds(pl.multiple_of(16 * jm1, 16), 16)], bs_b)
        nlt = plsc.all_reduce_population_count(bs_b[...] < x)[0]
        return jnp.where(jstar == 0, 0, jm1 * 16 + nlt)

    start = bpos(own_lo)
    end = bpos(own_hi)

    def wrowv(j):
        return [wt_b[j, pl.ds(c * 16, 16)] for c in range(4)]

    w0, w1, w2, w3, w4, w5, w6, w7, w8 = [wrowv(j) for j in range(9)]
    r0 = [w0[c] + w4[c] + w7[c] for c in range(4)]
    r1 = [w1[c] + w5[c] + w8[c] for c in range(4)]
    r2 = [w2[c] + w6[c] for c in range(4)]
    r3 = w3
    r4 = [-VX * w7[c] for c in range(4)]
    r5 = [-VY * w8[c] for c in range(4)]
    bias = [-X_OFFSET * w7[c] - Y_OFFSET * w8[c] for c in range(4)]

    col = [jnp.full((16,), cc, jnp.int32) for cc in range(4)]

    def piece_body(_, st):
        q = st[0]
        ofs = jnp.minimum(q - lax.rem(q, 16), N - P)
        ofs = pl.multiple_of(ofs, 16)
        pltpu.sync_copy(points_hbm.at[pl.ds(ofs, P)], pts_b)
        pltpu.sync_copy(ids_hbm.at[pl.ds(ofs, P)], ids_b.at[pl.ds(0, P)])
        pltpu.sync_copy(grid_hbm.at[pl.ds(ofs, P)], grd_b)
        qhi = jnp.minimum(ofs + P, end)
        pp = jnp.maximum(qhi - q, 0)
        base = q - ofs

        def it_body(k, s):
            (cur, sc, cntv, sxv, syv, szv, a0, a1, a2, a3) = s
            i = base + k
            vid = ids_b[pl.ds(i, 16)][0]
            iv = jnp.full((16,), 0, jnp.int32) + i
            p0v = plsc.load_gather(pts_b, [iv, col[0]])
            p1v = plsc.load_gather(pts_b, [iv, col[1]])
            p2v = plsc.load_gather(pts_b, [iv, col[2]])
            p3v = plsc.load_gather(pts_b, [iv, col[3]])
            g2v = plsc.load_gather(grd_b, [iv, col[2]]).astype(jnp.float32)
            g3v = plsc.load_gather(grd_b, [iv, col[3]]).astype(jnp.float32)
            A = [(bias[c] + p0v * r0[c]) + (p1v * r1[c] + p2v * r2[c])
                 + ((p3v * r3[c] + g3v * r4[c]) + g2v * r5[c])
                 for c in range(4)]

            is_new = vid != cur
            do_close = jnp.logical_and(is_new, cur >= 0)
            slot = lax.rem(sc, 16)

            @pl.when(do_close)
            def _():
                @pl.when(jnp.logical_and(slot == 0, sc > 0))
                def _():
                    def dr(j, _):
                        pltpu.make_async_copy(
                            stage_b.at[pl.ds(0, 64)],
                            out_hbm.at[pl.ds(0, 64)], sem).wait()
                        return 0

                    lax.fori_loop(0, 16, dr, 0)

                acc = (a0, a1, a2, a3)
                sbase = slot * 64
                for c in range(4):
                    m = ((sxv * w4[c] + syv * w5[c]) + szv * w6[c]) / cntv
                    stage_b[pl.ds(sbase + 16 * c, 16)] = jnp.maximum(
                        acc[c] - m, 0.0)
                pltpu.async_copy(
                    stage_b.at[pl.ds(pl.multiple_of(sbase, 64), 64)],
                    out_hbm.at[pl.ds(pl.multiple_of(cur * 64, 64), 64)],
                    sem)

            sc2 = jnp.where(do_close, sc + 1, sc)
            onev = jnp.ones((16,), jnp.float32)
            cntv2 = jnp.where(is_new, onev, cntv + onev)
            sxv2 = jnp.where(is_new, p0v, sxv + p0v)
            syv2 = jnp.where(is_new, p1v, syv + p1v)
            szv2 = jnp.where(is_new, p2v, szv + p2v)
            a0n = jnp.where(is_new, A[0], jnp.maximum(a0, A[0]))
            a1n = jnp.where(is_new, A[1], jnp.maximum(a1, A[1]))
            a2n = jnp.where(is_new, A[2], jnp.maximum(a2, A[2]))
            a3n = jnp.where(is_new, A[3], jnp.maximum(a3, A[3]))
            return (vid, sc2, cntv2, sxv2, syv2, szv2, a0n, a1n, a2n, a3n)

        inner = lax.fori_loop(0, pp, it_body, st[1:])
        return (q + pp,) + inner

    zrow16 = jnp.zeros((16,), jnp.float32)
    npieces = lax.div(end - start, P - 16) + 2
    init = (start, jnp.int32(-1), jnp.int32(0),
            jnp.ones((16,), jnp.float32), zrow16, zrow16, zrow16,
            zrow16, zrow16, zrow16, zrow16)
    fin = lax.fori_loop(0, npieces, piece_body, init)
    (_, cur_f, sc_f, cntv_f, sxv_f, syv_f, szv_f, a0f, a1f, a2f, a3f) = fin

    @pl.when(end > start)
    def _():
        slot = lax.rem(sc_f, 16)

        @pl.when(jnp.logical_and(slot == 0, sc_f > 0))
        def _():
            def dr(j, _):
                pltpu.make_async_copy(stage_b.at[pl.ds(0, 64)],
                                      out_hbm.at[pl.ds(0, 64)], sem).wait()
                return 0

            lax.fori_loop(0, 16, dr, 0)

        sbase = slot * 64
        for c in range(4):
            acc = (a0f, a1f, a2f, a3f)[c]
            m = ((sxv_f * w4[c] + syv_f * w5[c]) + szv_f * w6[c]) / cntv_f
            stage_b[pl.ds(sbase + 16 * c, 16)] = jnp.maximum(acc - m, 0.0)
        pltpu.async_copy(
            stage_b.at[pl.ds(pl.multiple_of(sbase, 64), 64)],
            out_hbm.at[pl.ds(pl.multiple_of(cur_f * 64, 64), 64)], sem)

        sc2 = sc_f + 1
        out_cnt = lax.rem(sc2 - 1, 16) + 1

        def dr2(j, _):
            pltpu.make_async_copy(stage_b.at[pl.ds(0, 64)],
                                  out_hbm.at[pl.ds(0, 64)], sem).wait()
            return 0

        lax.fori_loop(0, out_cnt, dr2, 0)


def kernel(points, unq_inv, grid_ind, W):
    wt = jnp.transpose(W)
    out = _pfnet_kernel(points, unq_inv, grid_ind, wt)
    return out.reshape(V, D_OUT)


# speedup vs baseline: 3.0799x; 3.0799x over previous
import functools

import jax
import jax.numpy as jnp
from jax import lax
from jax.experimental import pallas as pl
from jax.experimental.pallas import tpu as pltpu
from jax.experimental.pallas import tpu_sc as plsc

N = 1600000
V = 50000
D_OUT = 64
VX = 0.2
VY = 0.2
X_OFFSET = VX / 2 + 0.0
Y_OFFSET = VY / 2 + (-40.0)

NC = 2
NS = 16
NW = NC * NS
C = N // NW
P = 10000
M = N // 16
BS_ITERS = 17

_mesh = plsc.VectorSubcoreMesh(core_axis_name="c", subcore_axis_name="s")


def _splat0(vec):
    return vec.at[jnp.zeros((16,), jnp.int32)].get(mode="promise_in_bounds")


@functools.partial(
    pl.kernel,
    out_type=jax.ShapeDtypeStruct((V * D_OUT,), jnp.float32),
    mesh=_mesh,
    scratch_types=[
        pltpu.VMEM((P + 16,), jnp.float32),
        pltpu.VMEM((P + 16,), jnp.float32),
        pltpu.VMEM((P + 16,), jnp.float32),
        pltpu.VMEM((P + 16,), jnp.float32),
        pltpu.VMEM((P + 16,), jnp.int32),
        pltpu.VMEM((P + 16,), jnp.int32),
        pltpu.VMEM((P + 32,), jnp.int32),
        pltpu.VMEM((9, 64), jnp.float32),
        pltpu.VMEM((16,), jnp.int32),
        pltpu.VMEM((16,), jnp.int32),
        pltpu.VMEM((16,), jnp.int32),
        pltpu.VMEM((16 * 64,), jnp.float32),
        pltpu.VMEM((16 * 64,), jnp.float32),
        pltpu.SemaphoreType.DMA,
        pltpu.SemaphoreType.DMA,
    ],
    compiler_params=pltpu.CompilerParams(use_tc_tiling_on_sc=False,
                                         needs_layout_passes=False),
)
def _pfnet_kernel(p0_hbm, p1_hbm, p2_hbm, p3_hbm, g2_hbm, g3_hbm,
                  ids_hbm, wt_hbm, out_hbm,
                  p0_b, p1_b, p2_b, p3_b, g2_b, g3_b, ids_b, wt_b,
                  lo_b, hi_b, bs_b, stage_b, zero_b, zsem, sem):
    w = lax.axis_index("c") * NS + lax.axis_index("s")

    pltpu.sync_copy(wt_hbm, wt_b)

    @pl.when(w > 0)
    def _():
        pltpu.sync_copy(ids_hbm.at[pl.ds(pl.multiple_of(w * C - 16, 16), 16)],
                        lo_b)

    @pl.when(w == 0)
    def _():
        lo_b[...] = jnp.full((16,), -1, jnp.int32)

    pltpu.sync_copy(
        ids_hbm.at[pl.ds(pl.multiple_of((w + 1) * C - 16, 16), 16)], hi_b)

    own_lo = lo_b[...][15] + 1
    own_hi = jnp.where(w == NW - 1, V, hi_b[...][15] + 1)

    for t in range(64):
        zero_b[pl.ds(16 * t, 16)] = jnp.zeros((16,), jnp.float32)

    nz = own_hi - own_lo
    ntile = lax.div(nz, 16)
    tail = nz - ntile * 16

    def ztile(t, _):
        dst = pl.multiple_of((own_lo + 16 * t) * 64, 64)
        pltpu.async_copy(zero_b, out_hbm.at[pl.ds(dst, 16 * 64)], zsem)
        return 0

    lax.fori_loop(0, ntile, ztile, 0)

    def zrow(j, _):
        dst = pl.multiple_of((own_lo + ntile * 16 + j) * 64, 64)
        pltpu.async_copy(zero_b.at[pl.ds(0, 64)],
                         out_hbm.at[pl.ds(dst, 64)], zsem)
        return 0

    lax.fori_loop(0, tail, zrow, 0)

    def zdrain_t(t, _):
        pltpu.make_async_copy(
            zero_b, out_hbm.at[pl.ds(pl.multiple_of(own_lo * 64, 64),
                                     16 * 64)], zsem).wait()
        return 0

    lax.fori_loop(0, ntile, zdrain_t, 0)

    def zdrain_r(j, _):
        pltpu.make_async_copy(
            zero_b.at[pl.ds(0, 64)],
            out_hbm.at[pl.ds(pl.multiple_of(own_lo * 64, 64), 64)],
            zsem).wait()
        return 0

    lax.fori_loop(0, tail, zdrain_r, 0)

    def bpos(x):
        def bs_body(_, s):
            lo, hi = s
            mid = lax.div(lo + hi, 2)
            pltpu.sync_copy(
                ids_hbm.at[pl.ds(pl.multiple_of(16 * mid, 16), 16)], bs_b)
            ge = bs_b[...][0] >= x
            lo2 = jnp.where(jnp.logical_and(lo < hi, jnp.logical_not(ge)),
                            mid + 1, lo)
            hi2 = jnp.where(jnp.logical_and(lo < hi, ge), mid, hi)
            return lo2, hi2

        jstar, _ = lax.fori_loop(0, BS_ITERS, bs_body,
                                 (jnp.int32(0), jnp.int32(M)))
        jm1 = jnp.maximum(jstar - 1, 0)
        pltpu.sync_copy(
            ids_hbm.at[pl.ds(pl.multiple_of(16 * jm1, 16), 16)], bs_b)
        nlt = plsc.all_reduce_population_count(bs_b[...] < x)[0]
        return jnp.where(jstar == 0, 0, jm1 * 16 + nlt)

    start = bpos(own_lo)
    end = bpos(own_hi)

    def wrowv(j):
        return [wt_b[j, pl.ds(c * 16, 16)] for c in range(4)]

    w0, w1, w2, w3, w4, w5, w6, w7, w8 = [wrowv(j) for j in range(9)]
    r0 = [w0[c] + w4[c] + w7[c] for c in range(4)]
    r1 = [w1[c] + w5[c] + w8[c] for c in range(4)]
    r2 = [w2[c] + w6[c] for c in range(4)]
    r3 = w3
    r4 = [-VX * w7[c] for c in range(4)]
    r5 = [-VY * w8[c] for c in range(4)]
    bias = [-X_OFFSET * w7[c] - Y_OFFSET * w8[c] for c in range(4)]

    def piece_body(_, st):
        q = st[0]
        ofs = jnp.minimum(q - lax.rem(q, 16), N - P)
        ofs = pl.multiple_of(ofs, 16)
        for src, dst in ((p0_hbm, p0_b), (p1_hbm, p1_b), (p2_hbm, p2_b),
                         (p3_hbm, p3_b), (g2_hbm, g2_b), (g3_hbm, g3_b)):
            pltpu.sync_copy(src.at[pl.ds(ofs, P)], dst.at[pl.ds(0, P)])
        pltpu.sync_copy(ids_hbm.at[pl.ds(ofs, P)], ids_b.at[pl.ds(0, P)])
        qhi = jnp.minimum(ofs + P, end)
        pp = jnp.maximum(qhi - q, 0)
        base = q - ofs

        def it_body(k, s):
            (cur, sc, cntv, sxv, syv, szv, a0, a1, a2, a3) = s
            i = base + k
            vid = ids_b[pl.ds(i, 16)][0]
            p0v = _splat0(p0_b[pl.ds(i, 16)])
            p1v = _splat0(p1_b[pl.ds(i, 16)])
            p2v = _splat0(p2_b[pl.ds(i, 16)])
            p3v = _splat0(p3_b[pl.ds(i, 16)])
            g2v = _splat0(g2_b[pl.ds(i, 16)]).astype(jnp.float32)
            g3v = _splat0(g3_b[pl.ds(i, 16)]).astype(jnp.float32)
            A = [(bias[c] + p0v * r0[c]) + (p1v * r1[c] + p2v * r2[c])
                 + ((p3v * r3[c] + g3v * r4[c]) + g2v * r5[c])
                 for c in range(4)]

            is_new = vid != cur
            do_close = jnp.logical_and(is_new, cur >= 0)
            slot = lax.rem(sc, 16)

            @pl.when(do_close)
            def _():
                @pl.when(jnp.logical_and(slot == 0, sc > 0))
                def _():
                    def dr(j, _):
                        pltpu.make_async_copy(
                            stage_b.at[pl.ds(0, 64)],
                            out_hbm.at[pl.ds(0, 64)], sem).wait()
                        return 0

                    lax.fori_loop(0, 16, dr, 0)

                acc = (a0, a1, a2, a3)
                sbase = slot * 64
                for c in range(4):
                    m = ((sxv * w4[c] + syv * w5[c]) + szv * w6[c]) / cntv
                    stage_b[pl.ds(sbase + 16 * c, 16)] = jnp.maximum(
                        acc[c] - m, 0.0)
                pltpu.async_copy(
                    stage_b.at[pl.ds(pl.multiple_of(sbase, 64), 64)],
                    out_hbm.at[pl.ds(pl.multiple_of(cur * 64, 64), 64)],
                    sem)

            sc2 = jnp.where(do_close, sc + 1, sc)
            onev = jnp.ones((16,), jnp.float32)
            cntv2 = jnp.where(is_new, onev, cntv + onev)
            sxv2 = jnp.where(is_new, p0v, sxv + p0v)
            syv2 = jnp.where(is_new, p1v, syv + p1v)
            szv2 = jnp.where(is_new, p2v, szv + p2v)
            a0n = jnp.where(is_new, A[0], jnp.maximum(a0, A[0]))
            a1n = jnp.where(is_new, A[1], jnp.maximum(a1, A[1]))
            a2n = jnp.where(is_new, A[2], jnp.maximum(a2, A[2]))
            a3n = jnp.where(is_new, A[3], jnp.maximum(a3, A[3]))
            return (vid, sc2, cntv2, sxv2, syv2, szv2, a0n, a1n, a2n, a3n)

        inner = lax.fori_loop(0, pp, it_body, st[1:])
        return (q + pp,) + inner

    zrow16 = jnp.zeros((16,), jnp.float32)
    npieces = lax.div(end - start, P - 16) + 2
    init = (start, jnp.int32(-1), jnp.int32(0),
            jnp.ones((16,), jnp.float32), zrow16, zrow16, zrow16,
            zrow16, zrow16, zrow16, zrow16)
    fin = lax.fori_loop(0, npieces, piece_body, init)
    (_, cur_f, sc_f, cntv_f, sxv_f, syv_f, szv_f, a0f, a1f, a2f, a3f) = fin

    @pl.when(end > start)
    def _():
        slot = lax.rem(sc_f, 16)

        @pl.when(jnp.logical_and(slot == 0, sc_f > 0))
        def _():
            def dr(j, _):
                pltpu.make_async_copy(stage_b.at[pl.ds(0, 64)],
                                      out_hbm.at[pl.ds(0, 64)], sem).wait()
                return 0

            lax.fori_loop(0, 16, dr, 0)

        sbase = slot * 64
        for c in range(4):
            acc = (a0f, a1f, a2f, a3f)[c]
            m = ((sxv_f * w4[c] + syv_f * w5[c]) + szv_f * w6[c]) / cntv_f
            stage_b[pl.ds(sbase + 16 * c, 16)] = jnp.maximum(acc - m, 0.0)
        pltpu.async_copy(
            stage_b.at[pl.ds(pl.multiple_of(sbase, 64), 64)],
            out_hbm.at[pl.ds(pl.multiple_of(cur_f * 64, 64), 64)], sem)

        out_cnt = lax.rem(sc_f, 16) + 1

        def dr2(j, _):
            pltpu.make_async_copy(stage_b.at[pl.ds(0, 64)],
                                  out_hbm.at[pl.ds(0, 64)], sem).wait()
            return 0

        lax.fori_loop(0, out_cnt, dr2, 0)


def kernel(points, unq_inv, grid_ind, W):
    wt = jnp.transpose(W)
    out = _pfnet_kernel(points[:, 0], points[:, 1], points[:, 2],
                        points[:, 3], grid_ind[:, 2], grid_ind[:, 3],
                        unq_inv, wt)
    return out.reshape(V, D_OUT)
